# baseline (speedup 1.0000x reference)
import functools

import jax
import jax.numpy as jnp
from jax import lax
from jax.experimental import pallas as pl
from jax.experimental.pallas import tpu as pltpu
from jax.experimental.pallas import tpu_sc as plsc

B = 16384
D = 64
NC = 2
NS = 16
NW = NC * NS
BPW = B // NW
CH = 128
NCH = BPW // CH

_sc_mesh = plsc.VectorSubcoreMesh(core_axis_name="c", subcore_axis_name="s")


@functools.partial(
    pl.kernel,
    mesh=_sc_mesh,
    compiler_params=pltpu.CompilerParams(use_tc_tiling_on_sc=False),
    out_type=[jax.ShapeDtypeStruct((B, D), jnp.float32) for _ in range(4)],
    scratch_types=[
        pltpu.VMEM((NCH, CH), jnp.int32),
        pltpu.VMEM((NCH, CH), jnp.int32),
        pltpu.VMEM((BPW, D), jnp.float32),
        pltpu.VMEM((BPW, D), jnp.float32),
        pltpu.SemaphoreType.DMA,
        pltpu.SemaphoreType.DMA,
    ],
)
def _sc_gather(users_hbm, items_hbm, t_um, t_im, t_ug, t_ig,
               o_um, o_im, o_ug, o_ig,
               uidx, iidx, buf_a, buf_b, sem_a, sem_b):
    wid = lax.axis_index("s") * NC + lax.axis_index("c")
    base = wid * BPW
    pltpu.sync_copy(users_hbm.at[pl.ds(wid * NCH, NCH)], uidx)
    pltpu.sync_copy(items_hbm.at[pl.ds(wid * NCH, NCH)], iidx)

    plan = ((t_um, uidx, o_um), (t_im, iidx, o_im),
            (t_ug, uidx, o_ug), (t_ig, iidx, o_ig))
    bufs = (buf_a, buf_b)
    sems = (sem_a, sem_b)

    handles = [None, None]
    for t, (tbl, idx, _) in enumerate(plan[:2]):
        handles[t] = [
            pltpu.async_copy(tbl.at[idx.at[c]], bufs[t].at[pl.ds(c * CH, CH)], sems[t])
            for c in range(NCH)
        ]
    for t, (_, _, out) in enumerate(plan):
        slot = t % 2
        for h in handles[slot]:
            h.wait()
        if t + 2 < len(plan):
            tbl, idx, _ = plan[t + 2]
            handles[slot] = [
                pltpu.async_copy(tbl.at[idx.at[c]], bufs[slot].at[pl.ds(c * CH, CH)], sems[slot])
                for c in range(NCH)
            ]
        pltpu.sync_copy(bufs[slot], out.at[pl.ds(base, BPW)])


BB = 2048


def _mlp_body(um, im, ug, ig, w1a, w1b, b1, w2, b2, w3, b3, wg, bg,
              wfa, wfb, bfv, out):
    f32 = jnp.float32
    h = jnp.dot(um[...], w1a[...], preferred_element_type=f32)
    h = h + jnp.dot(im[...], w1b[...], preferred_element_type=f32) + b1[...]
    h = jnp.maximum(h, 0.0)
    h = jnp.maximum(jnp.dot(h, w2[...], preferred_element_type=f32) + b2[...], 0.0)
    h = jnp.dot(h, w3[...], preferred_element_type=f32) + b3[...]
    g = ug[...] * ig[...]
    og = jnp.dot(g, wg[...], preferred_element_type=f32) + bg[...]
    o = jnp.sum(h * wfa[...], axis=1) + jnp.sum(og * wfb[...], axis=1)
    out[...] = o + bfv[0, 0]


def _mlp(um, im, ug, ig, w1a, w1b, b1, w2, b2, w3, b3, wg, bg, wfa, wfb, bfv):
    full = lambda shape: pl.BlockSpec(shape, lambda i: (0,) * len(shape))
    blk = pl.BlockSpec((BB, D), lambda i: (i, 0))
    return pl.pallas_call(
        _mlp_body,
        grid=(B // BB,),
        in_specs=[
            blk, blk, blk, blk,
            full((D, D)), full((D, D)), full((1, D)),
            full((D, D)), full((1, D)),
            full((D, D // 2)), full((1, D // 2)),
            full((D, D // 2)), full((1, D // 2)),
            full((1, D // 2)), full((1, D // 2)), full((1, 1)),
        ],
        out_specs=pl.BlockSpec((BB,), lambda i: (i,)),
        out_shape=jax.ShapeDtypeStruct((B,), jnp.float32),
    )(um, im, ug, ig, w1a, w1b, b1, w2, b2, w3, b3, wg, bg, wfa, wfb, bfv)


def kernel(users, items, ue_mlp, ie_mlp, ue_gmf, ie_gmf,
           W_gmf, b_gmf, W1, b1, W2, b2, W3, b3, Wf, bf):
    users2d = users.astype(jnp.int32).reshape(B // CH, CH)
    items2d = items.astype(jnp.int32).reshape(B // CH, CH)
    um, im, ug, ig = _sc_gather(users2d, items2d, ue_mlp, ie_mlp, ue_gmf, ie_gmf)
    return _mlp(
        um, im, ug, ig,
        W1[:, :D].T, W1[:, D:].T, b1.reshape(1, D),
        W2.T, b2.reshape(1, D),
        W3.T, b3.reshape(1, D // 2),
        W_gmf.T, b_gmf.reshape(1, D // 2),
        Wf[:, :D // 2], Wf[:, D // 2:], bf.reshape(1, 1),
    )


# speedup vs baseline: 2.3089x; 2.3089x over previous
import functools

import jax
import jax.numpy as jnp
from jax import lax
from jax.experimental import pallas as pl
from jax.experimental.pallas import tpu as pltpu
from jax.experimental.pallas import tpu_sc as plsc

B = 16384
D = 64
V = 1000000
NC = 2
NS = 16
NW = NC * NS
BPW = B // NW
CH = 128
NCH = BPW // CH

TW = 8192
TG = (V + TW - 1) // TW

_sc_mesh = plsc.VectorSubcoreMesh(core_axis_name="c", subcore_axis_name="s")


def _transpose_body(ta, tb, out):
    out[:, :D] = jnp.swapaxes(ta[...], 0, 1)
    out[:, D:] = jnp.swapaxes(tb[...], 0, 1)


def _merge_transpose(ta_t, tb_t):
    return pl.pallas_call(
        _transpose_body,
        grid=(TG,),
        in_specs=[
            pl.BlockSpec((D, TW), lambda i: (0, i)),
            pl.BlockSpec((D, TW), lambda i: (0, i)),
        ],
        out_specs=pl.BlockSpec((TW, 2 * D), lambda i: (i, 0)),
        out_shape=jax.ShapeDtypeStruct((V, 2 * D), jnp.float32),
    )(ta_t, tb_t)


@functools.partial(
    pl.kernel,
    mesh=_sc_mesh,
    out_type=[jax.ShapeDtypeStruct((B, 2 * D), jnp.float32) for _ in range(2)],
    scratch_types=[
        pltpu.VMEM((NCH, CH), jnp.int32),
        pltpu.VMEM((BPW, 2 * D), jnp.float32),
        pltpu.SemaphoreType.DMA,
    ],
)
def _sc_gather(users3d, items3d, user_tab, item_tab, o_u, o_i, idx, buf, sem):
    wid = lax.axis_index("s") * NC + lax.axis_index("c")
    base = wid * BPW
    for idx3d, tab, out in ((users3d, user_tab, o_u), (items3d, item_tab, o_i)):
        pltpu.sync_copy(idx3d.at[wid], idx)
        handles = [
            pltpu.async_copy(tab.at[idx.at[c]], buf.at[pl.ds(c * CH, CH)], sem)
            for c in range(NCH)
        ]
        for h in handles:
            h.wait()
        pltpu.sync_copy(buf, out.at[pl.ds(base, BPW)])


BB = 2048


def _mlp_body(eu, ei, w1a, w1b, b1, w2, b2, w3, b3, wg, bg, wfa, wfb, bfv, out):
    f32 = jnp.float32
    um, ug = eu[:, :D], eu[:, D:]
    im, ig = ei[:, :D], ei[:, D:]
    h = jnp.dot(um, w1a[...], preferred_element_type=f32)
    h = h + jnp.dot(im, w1b[...], preferred_element_type=f32) + b1[...]
    h = jnp.maximum(h, 0.0)
    h = jnp.maximum(jnp.dot(h, w2[...], preferred_element_type=f32) + b2[...], 0.0)
    h = jnp.dot(h, w3[...], preferred_element_type=f32) + b3[...]
    g = ug * ig
    og = jnp.dot(g, wg[...], preferred_element_type=f32) + bg[...]
    o = jnp.sum(h * wfa[...], axis=1) + jnp.sum(og * wfb[...], axis=1)
    out[...] = o + bfv[0, 0]


def _mlp(eu, ei, w1a, w1b, b1, w2, b2, w3, b3, wg, bg, wfa, wfb, bfv):
    full = lambda shape: pl.BlockSpec(shape, lambda i: (0,) * len(shape))
    blk = pl.BlockSpec((BB, 2 * D), lambda i: (i, 0))
    return pl.pallas_call(
        _mlp_body,
        grid=(B // BB,),
        in_specs=[
            blk, blk,
            full((D, D)), full((D, D)), full((1, D)),
            full((D, D)), full((1, D)),
            full((D, D // 2)), full((1, D // 2)),
            full((D, D // 2)), full((1, D // 2)),
            full((1, D // 2)), full((1, D // 2)), full((1, 1)),
        ],
        out_specs=pl.BlockSpec((BB,), lambda i: (i,)),
        out_shape=jax.ShapeDtypeStruct((B,), jnp.float32),
    )(eu, ei, w1a, w1b, b1, w2, b2, w3, b3, wg, bg, wfa, wfb, bfv)


def kernel(users, items, ue_mlp, ie_mlp, ue_gmf, ie_gmf,
           W_gmf, b_gmf, W1, b1, W2, b2, W3, b3, Wf, bf):
    users3d = users.astype(jnp.int32).reshape(NW, NCH, CH)
    items3d = items.astype(jnp.int32).reshape(NW, NCH, CH)
    user_tab = _merge_transpose(ue_mlp.T, ue_gmf.T)
    item_tab = _merge_transpose(ie_mlp.T, ie_gmf.T)
    eu, ei = _sc_gather(users3d, items3d, user_tab, item_tab)
    return _mlp(
        eu, ei,
        W1[:, :D].T, W1[:, D:].T, b1.reshape(1, D),
        W2.T, b2.reshape(1, D),
        W3.T, b3.reshape(1, D // 2),
        W_gmf.T, b_gmf.reshape(1, D // 2),
        Wf[:, :D // 2], Wf[:, D // 2:], bf.reshape(1, 1),
    )


# speedup vs baseline: 3.6960x; 1.6008x over previous
import functools

import jax
import jax.numpy as jnp
from jax import lax
from jax.experimental import pallas as pl
from jax.experimental.pallas import tpu as pltpu
from jax.experimental.pallas import tpu_sc as plsc

B = 16384
D = 64
V = 1000000
NC = 2
NS = 16
NW = NC * NS
BPW = B // NW
CH = 128
NCH = BPW // CH

TW = 8192
TG = (V + TW - 1) // TW

_sc_mesh = plsc.VectorSubcoreMesh(core_axis_name="c", subcore_axis_name="s")


def _pack_body(ta, tb, tc_, td, out):
    ident = (jax.lax.broadcasted_iota(jnp.int32, (D, D), 0)
             == jax.lax.broadcasted_iota(jnp.int32, (D, D), 1)
             ).astype(jnp.bfloat16)
    dn = (((0,), (0,)), ((), ()))
    cols = []
    for t in (ta, tb, tc_, td):
        x = jax.lax.dot_general(t[...].astype(jnp.bfloat16), ident, dn,
                                preferred_element_type=jnp.float32
                                ).astype(jnp.bfloat16)
        cols.append(jax.lax.bitcast_convert_type(x, jnp.uint16).astype(jnp.uint32))
    u = cols[0] | (cols[1] << 16)
    i = cols[2] | (cols[3] << 16)
    out[:, :D] = jax.lax.bitcast_convert_type(u, jnp.float32)
    out[:, D:] = jax.lax.bitcast_convert_type(i, jnp.float32)


def _mega_pack(ua_t, ub_t, ia_t, ib_t):
    blk = pl.BlockSpec((D, TW), lambda i: (0, i))
    return pl.pallas_call(
        _pack_body,
        grid=(TG,),
        in_specs=[blk, blk, blk, blk],
        out_specs=pl.BlockSpec((TW, 2 * D), lambda i: (i, 0)),
        out_shape=jax.ShapeDtypeStruct((V, 2 * D), jnp.float32),
    )(ua_t, ub_t, ia_t, ib_t)


@functools.partial(
    pl.kernel,
    mesh=_sc_mesh,
    out_type=[jax.ShapeDtypeStruct((B, 2 * D), jnp.float32) for _ in range(2)],
    scratch_types=[
        pltpu.VMEM((NCH, CH), jnp.int32),
        pltpu.VMEM((BPW, 2 * D), jnp.float32),
        pltpu.SemaphoreType.DMA,
    ],
)
def _sc_gather(users3d, items3d, tab, o_u, o_i, idx, buf, sem):
    wid = lax.axis_index("s") * NC + lax.axis_index("c")
    base = wid * BPW
    for idx3d, out in ((users3d, o_u), (items3d, o_i)):
        pltpu.sync_copy(idx3d.at[wid], idx)
        handles = [
            pltpu.async_copy(tab.at[idx.at[c]], buf.at[pl.ds(c * CH, CH)], sem)
            for c in range(NCH)
        ]
        for h in handles:
            h.wait()
        pltpu.sync_copy(buf, out.at[pl.ds(base, BPW)])


BB = 2048


def _unpack(packed):
    w = jax.lax.bitcast_convert_type(packed, jnp.uint32)
    lo = jax.lax.bitcast_convert_type((w & 0xFFFF).astype(jnp.uint16), jnp.bfloat16)
    hi = jax.lax.bitcast_convert_type((w >> 16).astype(jnp.uint16), jnp.bfloat16)
    return lo, hi


def _mlp_body(eu, ei, w1a, w1b, b1, w2, b2, w3, b3, wg, bg, wfa, wfb, bfv, out):
    f32 = jnp.float32
    um, ug = _unpack(eu[:, :D])
    im, ig = _unpack(ei[:, D:])
    h = jnp.dot(um, w1a[...], preferred_element_type=f32)
    h = h + jnp.dot(im, w1b[...], preferred_element_type=f32) + b1[...]
    h = jnp.maximum(h, 0.0)
    h = jnp.maximum(jnp.dot(h, w2[...], preferred_element_type=f32) + b2[...], 0.0)
    h = jnp.dot(h, w3[...], preferred_element_type=f32) + b3[...]
    g = ug.astype(f32) * ig.astype(f32)
    og = jnp.dot(g, wg[...], preferred_element_type=f32) + bg[...]
    o = jnp.sum(h * wfa[...], axis=1) + jnp.sum(og * wfb[...], axis=1)
    out[...] = o + bfv[0, 0]


def _mlp(eu, ei, w1a, w1b, b1, w2, b2, w3, b3, wg, bg, wfa, wfb, bfv):
    full = lambda shape: pl.BlockSpec(shape, lambda i: (0,) * len(shape))
    blk = pl.BlockSpec((BB, 2 * D), lambda i: (i, 0))
    return pl.pallas_call(
        _mlp_body,
        grid=(B // BB,),
        in_specs=[
            blk, blk,
            full((D, D)), full((D, D)), full((1, D)),
            full((D, D)), full((1, D)),
            full((D, D // 2)), full((1, D // 2)),
            full((D, D // 2)), full((1, D // 2)),
            full((1, D // 2)), full((1, D // 2)), full((1, 1)),
        ],
        out_specs=pl.BlockSpec((BB,), lambda i: (i,)),
        out_shape=jax.ShapeDtypeStruct((B,), jnp.float32),
    )(eu, ei, w1a, w1b, b1, w2, b2, w3, b3, wg, bg, wfa, wfb, bfv)


def kernel(users, items, ue_mlp, ie_mlp, ue_gmf, ie_gmf,
           W_gmf, b_gmf, W1, b1, W2, b2, W3, b3, Wf, bf):
    users3d = users.astype(jnp.int32).reshape(NW, NCH, CH)
    items3d = items.astype(jnp.int32).reshape(NW, NCH, CH)
    tab = _mega_pack(ue_mlp.T, ue_gmf.T, ie_mlp.T, ie_gmf.T)
    eu, ei = _sc_gather(users3d, items3d, tab)
    return _mlp(
        eu, ei,
        W1[:, :D].T.astype(jnp.bfloat16), W1[:, D:].T.astype(jnp.bfloat16),
        b1.reshape(1, D),
        W2.T, b2.reshape(1, D),
        W3.T, b3.reshape(1, D // 2),
        W_gmf.T, b_gmf.reshape(1, D // 2),
        Wf[:, :D // 2], Wf[:, D // 2:], bf.reshape(1, 1),
    )


# speedup vs baseline: 3.8599x; 1.0444x over previous
import functools

import jax
import jax.numpy as jnp
from jax import lax
from jax.experimental import pallas as pl
from jax.experimental.pallas import tpu as pltpu
from jax.experimental.pallas import tpu_sc as plsc

B = 16384
D = 64
V = 1000000
NC = 2
NS = 16
NW = NC * NS
BPW = B // NW
CH = 128
NCH = BPW // CH

TW = 16384
TG = (V + TW - 1) // TW

_sc_mesh = plsc.VectorSubcoreMesh(core_axis_name="c", subcore_axis_name="s")


def _pack_body(ta, tb, tc_, td, out):
    ident = (jax.lax.broadcasted_iota(jnp.int32, (D, D), 0)
             == jax.lax.broadcasted_iota(jnp.int32, (D, D), 1)
             ).astype(jnp.bfloat16)
    dn = (((0,), (0,)), ((), ()))
    cols = []
    for t in (ta, tb, tc_, td):
        x = jax.lax.dot_general(t[...].astype(jnp.bfloat16), ident, dn,
                                preferred_element_type=jnp.float32
                                ).astype(jnp.bfloat16)
        cols.append(jax.lax.bitcast_convert_type(x, jnp.uint16).astype(jnp.uint32))
    u = cols[0] | (cols[1] << 16)
    i = cols[2] | (cols[3] << 16)
    out[:, :D] = jax.lax.bitcast_convert_type(u, jnp.float32)
    out[:, D:] = jax.lax.bitcast_convert_type(i, jnp.float32)


def _mega_pack(ua_t, ub_t, ia_t, ib_t):
    blk = pl.BlockSpec((D, TW), lambda i: (0, i))
    return pl.pallas_call(
        _pack_body,
        grid=(TG,),
        in_specs=[blk, blk, blk, blk],
        out_specs=pl.BlockSpec((TW, 2 * D), lambda i: (i, 0)),
        out_shape=jax.ShapeDtypeStruct((V, 2 * D), jnp.float32),
    )(ua_t, ub_t, ia_t, ib_t)


@functools.partial(
    pl.kernel,
    mesh=_sc_mesh,
    out_type=[jax.ShapeDtypeStruct((B, 2 * D), jnp.float32) for _ in range(2)],
    scratch_types=[
        pltpu.VMEM((NCH, CH), jnp.int32),
        pltpu.VMEM((BPW, 2 * D), jnp.float32),
        pltpu.SemaphoreType.DMA,
    ],
)
def _sc_gather(users3d, items3d, tab, o_u, o_i, idx, buf, sem):
    wid = lax.axis_index("s") * NC + lax.axis_index("c")
    base = wid * BPW
    for idx3d, out in ((users3d, o_u), (items3d, o_i)):
        pltpu.sync_copy(idx3d.at[wid], idx)
        handles = [
            pltpu.async_copy(tab.at[idx.at[c]], buf.at[pl.ds(c * CH, CH)], sem)
            for c in range(NCH)
        ]
        for h in handles:
            h.wait()
        pltpu.sync_copy(buf, out.at[pl.ds(base, BPW)])


BB = 2048


def _unpack(packed):
    w = jax.lax.bitcast_convert_type(packed, jnp.uint32)
    lo = jax.lax.bitcast_convert_type((w & 0xFFFF).astype(jnp.uint16), jnp.bfloat16)
    hi = jax.lax.bitcast_convert_type((w >> 16).astype(jnp.uint16), jnp.bfloat16)
    return lo, hi


def _mlp_body(eu, ei, w1a, w1b, b1, w2, b2, w3, b3, wg, bg, wfa, wfb, bfv, out):
    f32 = jnp.float32
    um, ug = _unpack(eu[:, :D])
    im, ig = _unpack(ei[:, D:])
    h = jnp.dot(um, w1a[...], preferred_element_type=f32)
    h = h + jnp.dot(im, w1b[...], preferred_element_type=f32) + b1[...]
    h = jnp.maximum(h, 0.0)
    h = jnp.maximum(jnp.dot(h, w2[...], preferred_element_type=f32) + b2[...], 0.0)
    h = jnp.dot(h, w3[...], preferred_element_type=f32) + b3[...]
    g = ug.astype(f32) * ig.astype(f32)
    og = jnp.dot(g, wg[...], preferred_element_type=f32) + bg[...]
    o = jnp.sum(h * wfa[...], axis=1) + jnp.sum(og * wfb[...], axis=1)
    out[...] = o + bfv[0, 0]


def _mlp(eu, ei, w1a, w1b, b1, w2, b2, w3, b3, wg, bg, wfa, wfb, bfv):
    full = lambda shape: pl.BlockSpec(shape, lambda i: (0,) * len(shape))
    blk = pl.BlockSpec((BB, 2 * D), lambda i: (i, 0))
    return pl.pallas_call(
        _mlp_body,
        grid=(B // BB,),
        in_specs=[
            blk, blk,
            full((D, D)), full((D, D)), full((1, D)),
            full((D, D)), full((1, D)),
            full((D, D // 2)), full((1, D // 2)),
            full((D, D // 2)), full((1, D // 2)),
            full((1, D // 2)), full((1, D // 2)), full((1, 1)),
        ],
        out_specs=pl.BlockSpec((BB,), lambda i: (i,)),
        out_shape=jax.ShapeDtypeStruct((B,), jnp.float32),
    )(eu, ei, w1a, w1b, b1, w2, b2, w3, b3, wg, bg, wfa, wfb, bfv)


def kernel(users, items, ue_mlp, ie_mlp, ue_gmf, ie_gmf,
           W_gmf, b_gmf, W1, b1, W2, b2, W3, b3, Wf, bf):
    users3d = users.astype(jnp.int32).reshape(NW, NCH, CH)
    items3d = items.astype(jnp.int32).reshape(NW, NCH, CH)
    tab = _mega_pack(ue_mlp.T, ue_gmf.T, ie_mlp.T, ie_gmf.T)
    eu, ei = _sc_gather(users3d, items3d, tab)
    return _mlp(
        eu, ei,
        W1[:, :D].T.astype(jnp.bfloat16), W1[:, D:].T.astype(jnp.bfloat16),
        b1.reshape(1, D),
        W2.T, b2.reshape(1, D),
        W3.T, b3.reshape(1, D // 2),
        W_gmf.T, b_gmf.reshape(1, D // 2),
        Wf[:, :D // 2], Wf[:, D // 2:], bf.reshape(1, 1),
    )
